# baseline (speedup 1.0000x reference)
import jax
import jax.numpy as jnp
from jax import lax
from jax.experimental import pallas as pl
from jax.experimental.pallas import tpu as pltpu
from jax.experimental.pallas import tpu_sc as plsc

_info = plsc.get_sparse_core_info()
_NC, _NS, _L = _info.num_cores, _info.num_subcores, _info.num_lanes
_NW = _NC * _NS

_B = 4
_S = 2048
_EMB = 1024
_P_PER_W = _S // _NW
_CP = 8
_NCHUNK = _P_PER_W // _CP
_VECS = _EMB // _L
_NBUF = 6
_LA = 3


def _body(x_hbm, tok_hbm, pos_hbm, out_hbm,
          idx_all, tok0, tok1, tok2, tok3, tok4, tok5, pos_v,
          isem, gsem0, gsem1, gsem2, gsem3, gsem4, gsem5,
          osem0, osem1, osem2, osem3, osem4, osem5,
          psem):
    wid = lax.axis_index("s") * _NC + lax.axis_index("c")
    pos0 = wid * _P_PER_W
    tok = [tok0, tok1, tok2, tok3, tok4, tok5]
    gsem = [gsem0, gsem1, gsem2, gsem3, gsem4, gsem5]
    osem = [osem0, osem1, osem2, osem3, osem4, osem5]

    icps = [pltpu.async_copy(x_hbm.at[b, pl.ds(pos0, _P_PER_W)],
                             idx_all.at[b], isem) for b in range(_B)]
    pos_cp = pltpu.async_copy(pos_hbm.at[pl.ds(pos0, _P_PER_W)], pos_v, psem)
    for c in icps:
        c.wait()

    units = [(pc, b) for pc in range(_NCHUNK) for b in range(_B)]
    nu = len(units)
    g_desc = [None] * _NBUF
    o_desc = [None] * _NBUF

    def issue(u):
        slot = u % _NBUF
        pc, b = units[u]
        g_desc[slot] = pltpu.async_copy(
            tok_hbm.at[idx_all.at[b, pl.ds(pc * _CP, _CP)]],
            tok[slot], gsem[slot])

    for v in range(_LA):
        issue(v)
    pos_cp.wait()
    for u in range(nu):
        cur = u % _NBUF
        if u + _LA < nu:
            nxt = (u + _LA) % _NBUF
            if o_desc[nxt] is not None:
                o_desc[nxt].wait()
            issue(u + _LA)
        g_desc[cur].wait()
        pc, b = units[u]
        tv = tok[cur]
        roff = pc * _CP

        @pl.loop(0, _CP * _VECS, unroll=8)
        def _(i):
            r = i >> 6
            c = (i & (_VECS - 1)) * _L
            plsc.addupdate(tv.at[r, pl.ds(c, _L)],
                           pos_v[roff + r, pl.ds(c, _L)])

        p = pos0 + pc * _CP
        o_desc[cur] = pltpu.async_copy(tv, out_hbm.at[b, pl.ds(p, _CP)],
                                       osem[cur])
    for v in range(nu - _NBUF, nu):
        o_desc[v % _NBUF].wait()


@jax.jit
def _emb(x, token_table, pos_table):
    mesh = plsc.VectorSubcoreMesh(core_axis_name="c", subcore_axis_name="s")
    return pl.kernel(
        _body,
        out_type=jax.ShapeDtypeStruct((_B, _S, _EMB), jnp.float32),
        mesh=mesh,
        scratch_types=(
            [pltpu.VMEM((_B, _P_PER_W), jnp.int32)]
            + [pltpu.VMEM((_CP, _EMB), jnp.float32) for _ in range(_NBUF)]
            + [pltpu.VMEM((_P_PER_W, _EMB), jnp.float32)]
            + [pltpu.SemaphoreType.DMA for _ in range(2 * _NBUF + 2)]
        ),
    )(x, token_table, pos_table)


def kernel(x, token_table, pos_table):
    return _emb(x.astype(jnp.int32), token_table, pos_table)


# speedup vs baseline: 1.4027x; 1.4027x over previous
import jax
import jax.numpy as jnp
from jax import lax
from jax.experimental import pallas as pl
from jax.experimental.pallas import tpu as pltpu
from jax.experimental.pallas import tpu_sc as plsc

_info = plsc.get_sparse_core_info()
_NC, _NS, _L = _info.num_cores, _info.num_subcores, _info.num_lanes
_NW = _NC * _NS

_B = 4
_S = 2048
_EMB = 1024
_P_PER_W = _S // _NW
_CP = 8
_NCHUNK = _P_PER_W // _CP
_VECS = _EMB // _L
_NBUF = 6
_LA = 3


def _body(x_hbm, tok_hbm, pos_hbm, out_hbm,
          idx_all, tok0, tok1, tok2, tok3, tok4, tok5, pos_v,
          isem, gsem0, gsem1, gsem2, gsem3, gsem4, gsem5,
          osem0, osem1, osem2, osem3, osem4, osem5,
          psem):
    wid = lax.axis_index("s") * _NC + lax.axis_index("c")
    pos0 = wid * _P_PER_W
    tok = [tok0, tok1, tok2, tok3, tok4, tok5]
    gsem = [gsem0, gsem1, gsem2, gsem3, gsem4, gsem5]
    osem = [osem0, osem1, osem2, osem3, osem4, osem5]

    icps = [pltpu.async_copy(x_hbm.at[b, pl.ds(pos0, _P_PER_W)],
                             idx_all.at[b], isem) for b in range(_B)]
    pos_cp = pltpu.async_copy(pos_hbm.at[pl.ds(pos0, _P_PER_W)], pos_v, psem)
    for c in icps:
        c.wait()

    units = [(pc, b) for pc in range(_NCHUNK) for b in range(_B)]
    nu = len(units)
    g_desc = [None] * _NBUF
    o_desc = [None] * _NBUF

    def issue(u):
        slot = u % _NBUF
        pc, b = units[u]
        g_desc[slot] = pltpu.async_copy(
            tok_hbm.at[idx_all.at[b, pl.ds(pc * _CP, _CP)]],
            tok[slot], gsem[slot])

    for v in range(_LA):
        issue(v)
    pos_cp.wait()
    pending = None
    for u in range(nu):
        cur = u % _NBUF
        if u + _LA < nu:
            nxt = (u + _LA) % _NBUF
            if o_desc[nxt] is not None:
                o_desc[nxt].wait()
            issue(u + _LA)
        g_desc[cur].wait()
        pc, b = units[u]
        tv = tok[cur]
        roff = pc * _CP

        @plsc.parallel_loop(0, _CP * _VECS, unroll=8)
        def _(i):
            r = i >> 6
            c = (i & (_VECS - 1)) * _L
            plsc.addupdate(tv.at[r, pl.ds(c, _L)],
                           pos_v[roff + r, pl.ds(c, _L)])

        if pending is not None:
            ps, pb, pp = pending
            o_desc[ps] = pltpu.async_copy(
                tok[ps], out_hbm.at[pb, pl.ds(pp, _CP)], osem[ps])
        pending = (cur, b, pos0 + pc * _CP)
    ps, pb, pp = pending
    o_desc[ps] = pltpu.async_copy(tok[ps], out_hbm.at[pb, pl.ds(pp, _CP)],
                                  osem[ps])
    for v in range(nu - _NBUF, nu):
        o_desc[v % _NBUF].wait()


@jax.jit
def _emb(x, token_table, pos_table):
    mesh = plsc.VectorSubcoreMesh(core_axis_name="c", subcore_axis_name="s")
    return pl.kernel(
        _body,
        out_type=jax.ShapeDtypeStruct((_B, _S, _EMB), jnp.float32),
        mesh=mesh,
        scratch_types=(
            [pltpu.VMEM((_B, _P_PER_W), jnp.int32)]
            + [pltpu.VMEM((_CP, _EMB), jnp.float32) for _ in range(_NBUF)]
            + [pltpu.VMEM((_P_PER_W, _EMB), jnp.float32)]
            + [pltpu.SemaphoreType.DMA for _ in range(2 * _NBUF + 2)]
        ),
    )(x, token_table, pos_table)


def kernel(x, token_table, pos_table):
    return _emb(x.astype(jnp.int32), token_table, pos_table)


# speedup vs baseline: 1.4119x; 1.0066x over previous
import jax
import jax.numpy as jnp
from jax import lax
from jax.experimental import pallas as pl
from jax.experimental.pallas import tpu as pltpu
from jax.experimental.pallas import tpu_sc as plsc

_info = plsc.get_sparse_core_info()
_NC, _NS, _L = _info.num_cores, _info.num_subcores, _info.num_lanes
_NW = _NC * _NS

_B = 4
_S = 2048
_EMB = 1024
_P_PER_W = _S // _NW
_CP = 8
_NCHUNK = _P_PER_W // _CP
_VECS = _EMB // _L
_NBUF = 7
_LA = 4


def _body(x_hbm, tok_hbm, pos_hbm, out_hbm,
          idx_all, tok0, tok1, tok2, tok3, tok4, tok5, tok6, pos_v,
          isem, gsem0, gsem1, gsem2, gsem3, gsem4, gsem5, gsem6,
          osem0, osem1, osem2, osem3, osem4, osem5, osem6,
          psem):
    wid = lax.axis_index("s") * _NC + lax.axis_index("c")
    pos0 = wid * _P_PER_W
    tok = [tok0, tok1, tok2, tok3, tok4, tok5, tok6]
    gsem = [gsem0, gsem1, gsem2, gsem3, gsem4, gsem5, gsem6]
    osem = [osem0, osem1, osem2, osem3, osem4, osem5, osem6]

    icps = [pltpu.async_copy(x_hbm.at[b, pl.ds(pos0, _P_PER_W)],
                             idx_all.at[b], isem) for b in range(_B)]
    pos_cp = pltpu.async_copy(pos_hbm.at[pl.ds(pos0, _P_PER_W)], pos_v, psem)
    for c in icps:
        c.wait()

    units = [(pc, b) for pc in range(_NCHUNK) for b in range(_B)]
    nu = len(units)
    g_desc = [None] * _NBUF
    o_desc = [None] * _NBUF

    def issue(u):
        slot = u % _NBUF
        pc, b = units[u]
        g_desc[slot] = pltpu.async_copy(
            tok_hbm.at[idx_all.at[b, pl.ds(pc * _CP, _CP)]],
            tok[slot], gsem[slot])

    for v in range(_LA):
        issue(v)
    pos_cp.wait()
    pending = None
    for u in range(nu):
        cur = u % _NBUF
        if u + _LA < nu:
            nxt = (u + _LA) % _NBUF
            if o_desc[nxt] is not None:
                o_desc[nxt].wait()
            issue(u + _LA)
        g_desc[cur].wait()
        pc, b = units[u]
        tv = tok[cur]
        roff = pc * _CP

        @plsc.parallel_loop(0, _CP * _VECS, unroll=8)
        def _(i):
            r = i >> 6
            c = (i & (_VECS - 1)) * _L
            plsc.addupdate(tv.at[r, pl.ds(c, _L)],
                           pos_v[roff + r, pl.ds(c, _L)])

        if pending is not None:
            ps, pb, pp = pending
            o_desc[ps] = pltpu.async_copy(
                tok[ps], out_hbm.at[pb, pl.ds(pp, _CP)], osem[ps])
        pending = (cur, b, pos0 + pc * _CP)
    ps, pb, pp = pending
    o_desc[ps] = pltpu.async_copy(tok[ps], out_hbm.at[pb, pl.ds(pp, _CP)],
                                  osem[ps])
    for v in range(nu - _NBUF, nu):
        o_desc[v % _NBUF].wait()


@jax.jit
def _emb(x, token_table, pos_table):
    mesh = plsc.VectorSubcoreMesh(core_axis_name="c", subcore_axis_name="s")
    return pl.kernel(
        _body,
        out_type=jax.ShapeDtypeStruct((_B, _S, _EMB), jnp.float32),
        mesh=mesh,
        scratch_types=(
            [pltpu.VMEM((_B, _P_PER_W), jnp.int32)]
            + [pltpu.VMEM((_CP, _EMB), jnp.float32) for _ in range(_NBUF)]
            + [pltpu.VMEM((_P_PER_W, _EMB), jnp.float32)]
            + [pltpu.SemaphoreType.DMA for _ in range(2 * _NBUF + 2)]
        ),
    )(x, token_table, pos_table)


def kernel(x, token_table, pos_table):
    return _emb(x.astype(jnp.int32), token_table, pos_table)


# speedup vs baseline: 1.4347x; 1.0161x over previous
import jax
import jax.numpy as jnp
from jax import lax
from jax.experimental import pallas as pl
from jax.experimental.pallas import tpu as pltpu
from jax.experimental.pallas import tpu_sc as plsc

_info = plsc.get_sparse_core_info()
_NC, _NS, _L = _info.num_cores, _info.num_subcores, _info.num_lanes
_NW = _NC * _NS

_B = 4
_S = 2048
_EMB = 1024
_P_PER_W = _S // _NW
_CP = 8
_NCHUNK = _P_PER_W // _CP
_VECS = _EMB // _L
_NBUF = 7
_LA = 4


def _body(x_hbm, tok_hbm, pos_hbm, out_hbm,
          idx_all, tok0, tok1, tok2, tok3, tok4, tok5, tok6, pos_v,
          isem, gsem0, gsem1, gsem2, gsem3, gsem4, gsem5, gsem6,
          osem0, osem1, osem2, osem3, osem4, osem5, osem6,
          psem):
    wid = lax.axis_index("s") * _NC + lax.axis_index("c")
    pos0 = wid * _P_PER_W
    tok = [tok0, tok1, tok2, tok3, tok4, tok5, tok6]
    gsem = [gsem0, gsem1, gsem2, gsem3, gsem4, gsem5, gsem6]
    osem = [osem0, osem1, osem2, osem3, osem4, osem5, osem6]

    icps = [pltpu.async_copy(x_hbm.at[b, pl.ds(pos0, _P_PER_W)],
                             idx_all.at[b], isem) for b in range(_B)]
    pos_cps = [pltpu.async_copy(pos_hbm.at[pl.ds(pos0 + pc * _CP, _CP)],
                                pos_v.at[pl.ds(pc * _CP, _CP)], psem)
               for pc in range(_NCHUNK)]
    for c in icps:
        c.wait()

    units = [(pc, b) for pc in range(_NCHUNK) for b in range(_B)]
    nu = len(units)
    g_desc = [None] * _NBUF
    o_desc = [None] * _NBUF

    def issue(u):
        slot = u % _NBUF
        pc, b = units[u]
        g_desc[slot] = pltpu.async_copy(
            tok_hbm.at[idx_all.at[b, pl.ds(pc * _CP, _CP)]],
            tok[slot], gsem[slot])

    for v in range(_LA):
        issue(v)
    pending = None
    for u in range(nu):
        cur = u % _NBUF
        if u + _LA < nu:
            nxt = (u + _LA) % _NBUF
            if o_desc[nxt] is not None:
                o_desc[nxt].wait()
            issue(u + _LA)
        g_desc[cur].wait()
        pc, b = units[u]
        if b == 0:
            pos_cps[pc].wait()
        tv = tok[cur]
        roff = pc * _CP

        @plsc.parallel_loop(0, _CP * _VECS, unroll=8)
        def _(i):
            r = i >> 6
            c = (i & (_VECS - 1)) * _L
            plsc.addupdate(tv.at[r, pl.ds(c, _L)],
                           pos_v[roff + r, pl.ds(c, _L)])

        if pending is not None:
            ps, pb, pp = pending
            o_desc[ps] = pltpu.async_copy(
                tok[ps], out_hbm.at[pb, pl.ds(pp, _CP)], osem[ps])
        pending = (cur, b, pos0 + pc * _CP)
    ps, pb, pp = pending
    o_desc[ps] = pltpu.async_copy(tok[ps], out_hbm.at[pb, pl.ds(pp, _CP)],
                                  osem[ps])
    for v in range(nu - _NBUF, nu):
        o_desc[v % _NBUF].wait()


@jax.jit
def _emb(x, token_table, pos_table):
    mesh = plsc.VectorSubcoreMesh(core_axis_name="c", subcore_axis_name="s")
    return pl.kernel(
        _body,
        out_type=jax.ShapeDtypeStruct((_B, _S, _EMB), jnp.float32),
        mesh=mesh,
        scratch_types=(
            [pltpu.VMEM((_B, _P_PER_W), jnp.int32)]
            + [pltpu.VMEM((_CP, _EMB), jnp.float32) for _ in range(_NBUF)]
            + [pltpu.VMEM((_P_PER_W, _EMB), jnp.float32)]
            + [pltpu.SemaphoreType.DMA for _ in range(2 * _NBUF + 2)]
        ),
    )(x, token_table, pos_table)


def kernel(x, token_table, pos_table):
    return _emb(x.astype(jnp.int32), token_table, pos_table)
